# baseline (speedup 1.0000x reference)
import functools

import jax
import jax.numpy as jnp
from jax import lax
from jax.experimental import pallas as pl
from jax.experimental.pallas import tpu as pltpu
from jax.experimental.pallas import tpu_sc as plsc

_B, _C, _H, _W = 4, 96, 512, 512
_BH = 32
_TOTAL = _B * _H * _W
_NF = 1.0 / _C

_NWORK = 32
_LANES = 16
_CHUNK = _TOTAL // _NWORK
_HISTN = _C * _LANES


def _argmax_body(x_ref, idx_ref):
    x = x_ref[0]
    idx_ref[0] = jnp.argmax(x, axis=0).astype(jnp.int32)


def _sc_hist_body(idx_hbm, part_hbm, buf_ref, hist_ref):
    wid = lax.axis_index("c") * 16 + lax.axis_index("s")
    base = wid * _CHUNK

    def _zero(i, _):
        hist_ref[pl.ds(i * _LANES, _LANES)] = jnp.zeros((_LANES,), jnp.int32)
        return 0

    lax.fori_loop(0, _C, _zero, 0)

    pltpu.sync_copy(idx_hbm.at[pl.ds(base, _CHUNK)], buf_ref)

    lane = lax.iota(jnp.int32, _LANES)
    ones = jnp.ones((_LANES,), jnp.int32)

    def _scat(j, _):
        v = buf_ref[pl.ds(j * _LANES, _LANES)]
        plsc.addupdate_scatter(hist_ref, [v * _LANES + lane], ones)
        return 0

    lax.fori_loop(0, _CHUNK // _LANES, _scat, 0, unroll=8)

    pltpu.sync_copy(hist_ref, part_hbm.at[wid])


def _final_body(p_ref, loss_ref, dist_ref):
    p = jnp.sum(p_ref[...], axis=0, keepdims=True).astype(jnp.float32)
    row = lax.broadcasted_iota(jnp.int32, (_HISTN, _C), 0)
    col = lax.broadcasted_iota(jnp.int32, (_HISTN, _C), 1)
    sel = (lax.shift_right_logical(row, 4) == col).astype(jnp.float32)
    hist = jax.lax.dot_general(
        p, sel, (((1,), (0,)), ((), ())),
        preferred_element_type=jnp.float32,
    )
    dist = hist * (1.0 / _TOTAL)
    dist_ref[...] = dist
    z = (dist - _NF) * (1.0 / (1.0 - _NF))
    loss_ref[0, 0] = jnp.sqrt(jnp.sum(z * z))


def kernel(generated_masks):
    idx = pl.pallas_call(
        _argmax_body,
        grid=(_B, _H // _BH),
        in_specs=[pl.BlockSpec((1, _C, _BH, _W), lambda b, h: (b, 0, h, 0))],
        out_specs=pl.BlockSpec((1, _BH, _W), lambda b, h: (b, h, 0)),
        out_shape=jax.ShapeDtypeStruct((_B, _H, _W), jnp.int32),
    )(generated_masks)

    sc_hist = functools.partial(
        pl.kernel,
        mesh=plsc.VectorSubcoreMesh(core_axis_name="c", subcore_axis_name="s"),
        out_type=jax.ShapeDtypeStruct((_NWORK, _HISTN), jnp.int32),
        scratch_types=[
            pltpu.VMEM((_CHUNK,), jnp.int32),
            pltpu.VMEM((_HISTN,), jnp.int32),
        ],
        compiler_params=pltpu.CompilerParams(needs_layout_passes=False),
    )(_sc_hist_body)
    partials = sc_hist(idx.reshape(_TOTAL))

    loss2d, dist2d = pl.pallas_call(
        _final_body,
        out_specs=[
            pl.BlockSpec(memory_space=pltpu.SMEM),
            pl.BlockSpec((1, _C), lambda: (0, 0)),
        ],
        out_shape=[
            jax.ShapeDtypeStruct((1, 1), jnp.float32),
            jax.ShapeDtypeStruct((1, _C), jnp.float32),
        ],
    )(partials)
    return (loss2d[0, 0], dist2d[0])


# speedup vs baseline: 1.0783x; 1.0783x over previous
import functools

import jax
import jax.numpy as jnp
from jax import lax
from jax.experimental import pallas as pl
from jax.experimental.pallas import tpu as pltpu
from jax.experimental.pallas import tpu_sc as plsc

_B, _C, _H, _W = 4, 96, 512, 512
_BH = 32
_TOTAL = _B * _H * _W
_NF = 1.0 / _C

_NWORK = 32
_LANES = 16
_CHUNK = _TOTAL // _NWORK
_HISTN = _C * _LANES


def _argmax_body(x_ref, idx_ref):
    x = x_ref[0]
    idx_ref[0] = jnp.argmax(x, axis=0).astype(jnp.int32)


def _sc_hist_body(idx_hbm, part_hbm, buf_ref, hist_ref):
    wid = lax.axis_index("c") * 16 + lax.axis_index("s")
    base = wid * _CHUNK

    def _zero(i, _):
        hist_ref[pl.ds(i * _LANES, _LANES)] = jnp.zeros((_LANES,), jnp.int32)
        return 0

    lax.fori_loop(0, _C, _zero, 0)

    pltpu.sync_copy(idx_hbm.at[pl.ds(base, _CHUNK)], buf_ref)

    lane = lax.iota(jnp.int32, _LANES)
    ones = jnp.ones((_LANES,), jnp.int32)

    @plsc.parallel_loop(0, _CHUNK // _LANES, unroll=8)
    def _scat(j):
        v = buf_ref[pl.ds(j * _LANES, _LANES)]
        plsc.addupdate_scatter(hist_ref, [v * _LANES + lane], ones)

    pltpu.sync_copy(hist_ref, part_hbm.at[wid])


def _final_body(p_ref, loss_ref, dist_ref):
    p = jnp.sum(p_ref[...], axis=0, keepdims=True).astype(jnp.float32)
    row = lax.broadcasted_iota(jnp.int32, (_HISTN, _C), 0)
    col = lax.broadcasted_iota(jnp.int32, (_HISTN, _C), 1)
    sel = (lax.shift_right_logical(row, 4) == col).astype(jnp.float32)
    hist = jax.lax.dot_general(
        p, sel, (((1,), (0,)), ((), ())),
        preferred_element_type=jnp.float32,
    )
    dist = hist * (1.0 / _TOTAL)
    dist_ref[...] = dist
    z = (dist - _NF) * (1.0 / (1.0 - _NF))
    loss_ref[0, 0] = jnp.sqrt(jnp.sum(z * z))


def kernel(generated_masks):
    idx = pl.pallas_call(
        _argmax_body,
        grid=(_B, _H // _BH),
        in_specs=[pl.BlockSpec((1, _C, _BH, _W), lambda b, h: (b, 0, h, 0))],
        out_specs=pl.BlockSpec((1, _BH, _W), lambda b, h: (b, h, 0)),
        out_shape=jax.ShapeDtypeStruct((_B, _H, _W), jnp.int32),
    )(generated_masks)

    sc_hist = functools.partial(
        pl.kernel,
        mesh=plsc.VectorSubcoreMesh(core_axis_name="c", subcore_axis_name="s"),
        out_type=jax.ShapeDtypeStruct((_NWORK, _HISTN), jnp.int32),
        scratch_types=[
            pltpu.VMEM((_CHUNK,), jnp.int32),
            pltpu.VMEM((_HISTN,), jnp.int32),
        ],
        compiler_params=pltpu.CompilerParams(needs_layout_passes=False),
    )(_sc_hist_body)
    partials = sc_hist(idx.reshape(_TOTAL))

    loss2d, dist2d = pl.pallas_call(
        _final_body,
        out_specs=[
            pl.BlockSpec(memory_space=pltpu.SMEM),
            pl.BlockSpec((1, _C), lambda: (0, 0)),
        ],
        out_shape=[
            jax.ShapeDtypeStruct((1, 1), jnp.float32),
            jax.ShapeDtypeStruct((1, _C), jnp.float32),
        ],
    )(partials)
    return (loss2d[0, 0], dist2d[0])


# speedup vs baseline: 1.2187x; 1.1302x over previous
import jax
import jax.numpy as jnp
from jax.experimental import pallas as pl
from jax.experimental.pallas import tpu as pltpu

_B, _C, _H, _W = 4, 96, 512, 512
_BH = 64
_TOTAL = _B * _H * _W
_NF = 1.0 / _C


def _body(x_ref, loss_ref, dist_ref, acc_ref):
    step = pl.program_id(0) * pl.num_programs(1) + pl.program_id(1)
    nsteps = pl.num_programs(0) * pl.num_programs(1)

    @pl.when(step == 0)
    def _init():
        acc_ref[...] = jnp.zeros_like(acc_ref)

    x = x_ref[0]
    idx = jnp.argmax(x, axis=0).astype(jnp.int32)
    classes = jax.lax.broadcasted_iota(jnp.int32, (_C, _BH, _W), 0)
    onehot = (idx[None, :, :] == classes).astype(jnp.float32)
    acc_ref[...] += jnp.sum(onehot, axis=1)

    @pl.when(step == nsteps - 1)
    def _fin():
        hist = jnp.sum(acc_ref[...], axis=1, keepdims=True)
        dist = hist * (1.0 / _TOTAL)
        dist_ref[...] = dist
        z = (dist - _NF) * (1.0 / (1.0 - _NF))
        loss_ref[0, 0] = jnp.sqrt(jnp.sum(z * z))


def kernel(generated_masks):
    loss2d, dist2d = pl.pallas_call(
        _body,
        grid=(_B, _H // _BH),
        in_specs=[
            pl.BlockSpec((1, _C, _BH, _W), lambda b, h: (b, 0, h, 0)),
        ],
        out_specs=[
            pl.BlockSpec(memory_space=pltpu.SMEM),
            pl.BlockSpec((_C, 1), lambda b, h: (0, 0)),
        ],
        out_shape=[
            jax.ShapeDtypeStruct((1, 1), jnp.float32),
            jax.ShapeDtypeStruct((_C, 1), jnp.float32),
        ],
        scratch_shapes=[pltpu.VMEM((_C, _W), jnp.float32)],
    )(generated_masks)
    return (loss2d[0, 0], dist2d[:, 0])


# speedup vs baseline: 1.2633x; 1.0366x over previous
import jax
import jax.numpy as jnp
from jax.experimental import pallas as pl
from jax.experimental.pallas import tpu as pltpu

_B, _C, _H, _W = 4, 96, 512, 512
_BH = 128
_TOTAL = _B * _H * _W
_NF = 1.0 / _C


def _body(x_ref, loss_ref, dist_ref, acc_ref):
    step = pl.program_id(0) * pl.num_programs(1) + pl.program_id(1)
    nsteps = pl.num_programs(0) * pl.num_programs(1)

    @pl.when(step == 0)
    def _init():
        acc_ref[...] = jnp.zeros_like(acc_ref)

    x = x_ref[0]
    idx = jnp.argmax(x, axis=0).astype(jnp.int32)
    classes = jax.lax.broadcasted_iota(jnp.int32, (_C, _BH, _W), 0)
    onehot = (idx[None, :, :] == classes).astype(jnp.float32)
    acc_ref[...] += jnp.sum(onehot, axis=1)

    @pl.when(step == nsteps - 1)
    def _fin():
        hist = jnp.sum(acc_ref[...], axis=1, keepdims=True)
        dist = hist * (1.0 / _TOTAL)
        dist_ref[...] = dist
        z = (dist - _NF) * (1.0 / (1.0 - _NF))
        loss_ref[0, 0] = jnp.sqrt(jnp.sum(z * z))


def kernel(generated_masks):
    loss2d, dist2d = pl.pallas_call(
        _body,
        grid=(_B, _H // _BH),
        in_specs=[
            pl.BlockSpec((1, _C, _BH, _W), lambda b, h: (b, 0, h, 0)),
        ],
        out_specs=[
            pl.BlockSpec(memory_space=pltpu.SMEM),
            pl.BlockSpec((_C, 1), lambda b, h: (0, 0)),
        ],
        out_shape=[
            jax.ShapeDtypeStruct((1, 1), jnp.float32),
            jax.ShapeDtypeStruct((_C, 1), jnp.float32),
        ],
        scratch_shapes=[pltpu.VMEM((_C, _W), jnp.float32)],
    )(generated_masks)
    return (loss2d[0, 0], dist2d[:, 0])


# speedup vs baseline: 1.2725x; 1.0073x over previous
import jax
import jax.numpy as jnp
from jax.experimental import pallas as pl
from jax.experimental.pallas import tpu as pltpu

_B, _C, _H, _W = 4, 96, 512, 512
_BH = 64
_TOTAL = _B * _H * _W
_NF = 1.0 / _C


def _body(x_ref, loss_ref, dist_ref, acc_ref):
    step = pl.program_id(0) * pl.num_programs(1) + pl.program_id(1)
    nsteps = pl.num_programs(0) * pl.num_programs(1)

    @pl.when(step == 0)
    def _init():
        acc_ref[...] = jnp.zeros_like(acc_ref)

    x = x_ref[0]
    idx = jnp.argmax(x, axis=0).astype(jnp.int32)
    classes = jax.lax.broadcasted_iota(jnp.int32, (_C, _BH, _W), 0)
    onehot = (idx[None, :, :] == classes).astype(jnp.float32)
    part = jnp.sum(onehot.reshape(_C, _BH // 8, 8, _W), axis=1)
    acc_ref[...] += part

    @pl.when(step == nsteps - 1)
    def _fin():
        hist = jnp.sum(acc_ref[...], axis=(1, 2), keepdims=True)[:, 0, :]
        dist = hist * (1.0 / _TOTAL)
        dist_ref[...] = dist
        z = (dist - _NF) * (1.0 / (1.0 - _NF))
        loss_ref[0, 0] = jnp.sqrt(jnp.sum(z * z))


def kernel(generated_masks):
    loss2d, dist2d = pl.pallas_call(
        _body,
        grid=(_B, _H // _BH),
        in_specs=[
            pl.BlockSpec((1, _C, _BH, _W), lambda b, h: (b, 0, h, 0)),
        ],
        out_specs=[
            pl.BlockSpec(memory_space=pltpu.SMEM),
            pl.BlockSpec((_C, 1), lambda b, h: (0, 0)),
        ],
        out_shape=[
            jax.ShapeDtypeStruct((1, 1), jnp.float32),
            jax.ShapeDtypeStruct((_C, 1), jnp.float32),
        ],
        scratch_shapes=[pltpu.VMEM((_C, 8, _W), jnp.float32)],
    )(generated_masks)
    return (loss2d[0, 0], dist2d[:, 0])
